# Initial kernel scaffold; baseline (speedup 1.0000x reference)
import jax
import jax.numpy as jnp
from jax.experimental import pallas as pl


def kernel(ids, values, remaps, emb0, emb1, emb2, emb3):
    raise NotImplementedError("write your pallas kernel here")


# speedup vs baseline: 18.2130x; 18.2130x over previous
import jax
import jax.numpy as jnp
from jax import lax
from jax.experimental import pallas as pl
from jax.experimental.pallas import tpu as pltpu
from jax.experimental.pallas import tpu_sc as plsc

V = 100000
D = 64
NL = 4
B = 4096
T = 50
N = B * T

NC, NS, LANES = 2, 16, 16
NW = NC * NS
PER_W = N // NW
CHUNK = 256
NCHUNK = PER_W // CHUNK

LEVEL_SIZES = (100, 1000, 10000, 100000)
OFFS = (0, 100, 1100, 11100)
TOT = 111100
LVL_MULT = (1.0, 0.5, 1.0 / 3.0, 0.25)


def _norm_body(x_ref, o_ref):
    x = x_ref[...]
    s = jnp.sum(x * x, axis=1, keepdims=True)
    inv = lax.rsqrt(jnp.maximum(s, 1e-36))
    o_ref[...] = x * inv


def _normalize_table(embcat):
    R = 2048
    grid = (TOT + R - 1) // R
    return pl.pallas_call(
        _norm_body,
        grid=(grid,),
        in_specs=[pl.BlockSpec((R, D), lambda i: (i, 0))],
        out_specs=pl.BlockSpec((R, D), lambda i: (i, 0)),
        out_shape=jax.ShapeDtypeStruct((TOT, D), jnp.float32),
    )(embcat)


def _sc_body(ids_h, vals_h, rem0_h, rem1_h, rem2_h, rem3_h, tab_h, out_h,
             ids_v, vals_v, idl0, idl1, idl2, idl3,
             r0, r1, r2, r3, out_v, sem):
    cid = lax.axis_index("c")
    sid = lax.axis_index("s")
    wid = sid * NC + cid
    idls = (idl0, idl1, idl2, idl3)
    rows = (r0, r1, r2, r3)
    rems = (rem0_h, rem1_h, rem2_h, rem3_h)

    def chunk_body(ci, carry):
        base = wid * PER_W + ci * CHUNK
        pltpu.sync_copy(ids_h.at[pl.ds(base, CHUNK)], ids_v)
        cp_vals = pltpu.async_copy(vals_h.at[pl.ds(base, CHUNK)], vals_v, sem)
        cp_rem = [pltpu.async_copy(rems[l].at[ids_v], idls[l], sem)
                  for l in range(NL)]
        for cp in cp_rem:
            cp.wait()

        cps = [pltpu.async_copy(tab_h.at[idls[l]], rows[l], sem)
               for l in range(NL)]
        cp_vals.wait()
        for cp in cps:
            cp.wait()

        def group(g, c):
            gsl = pl.ds(g * LANES, LANES)
            i0 = idl0[gsl]
            i1 = idl1[gsl]
            i2 = idl2[gsl]
            i3 = idl3[gsl]
            val = vals_v[gsl]
            one = jnp.int32(1)
            z0 = jnp.minimum(jnp.abs(i0 - OFFS[0]), one).astype(jnp.float32)
            z1 = jnp.minimum(jnp.abs(i1 - OFFS[1]), one).astype(jnp.float32)
            z2 = jnp.minimum(jnp.abs(i2 - OFFS[2]), one).astype(jnp.float32)
            z3 = jnp.minimum(jnp.abs(i3 - OFFS[3]), one).astype(jnp.float32)
            m3 = 1.0 - z3
            m2 = 1.0 - z2
            m1 = 1.0 - z1
            sel3 = z3
            sel2 = z2 * m3
            sel1 = z1 * (m2 * m3)
            sel0 = z0 * (m1 * m2 * m3)
            vm1 = val - 1.0
            g0v = (1.0 + sel0 * vm1) * LVL_MULT[0]
            g1v = (1.0 + sel1 * vm1) * LVL_MULT[1]
            g2v = (1.0 + sel2 * vm1) * LVL_MULT[2]
            g3v = (1.0 + sel3 * vm1) * LVL_MULT[3]
            for k in range(LANES):
                t = g * LANES + k
                g0 = g0v[k]
                g1 = g1v[k]
                g2 = g2v[k]
                g3 = g3v[k]
                for j in range(D // LANES):
                    sl = pl.ds(j * LANES, LANES)
                    out_v[t, sl] = (g0 * r0[t, sl] + g1 * r1[t, sl]
                                    + g2 * r2[t, sl] + g3 * r3[t, sl])
            return c

        lax.fori_loop(0, CHUNK // LANES, group, 0)
        pltpu.sync_copy(out_v, out_h.at[pl.ds(base, CHUNK)])
        return carry

    lax.fori_loop(0, NCHUNK, chunk_body, 0)


_SC_SCRATCH = [
    pltpu.VMEM((CHUNK,), jnp.int32),
    pltpu.VMEM((CHUNK,), jnp.float32),
    pltpu.VMEM((CHUNK,), jnp.int32),
    pltpu.VMEM((CHUNK,), jnp.int32),
    pltpu.VMEM((CHUNK,), jnp.int32),
    pltpu.VMEM((CHUNK,), jnp.int32),
    pltpu.VMEM((CHUNK, D), jnp.float32),
    pltpu.VMEM((CHUNK, D), jnp.float32),
    pltpu.VMEM((CHUNK, D), jnp.float32),
    pltpu.VMEM((CHUNK, D), jnp.float32),
    pltpu.VMEM((CHUNK, D), jnp.float32),
    pltpu.SemaphoreType.DMA,
]

_sc_call = pl.kernel(
    _sc_body,
    out_type=jax.ShapeDtypeStruct((N, D), jnp.float32),
    mesh=plsc.VectorSubcoreMesh(core_axis_name="c", subcore_axis_name="s"),
    scratch_types=_SC_SCRATCH,
    compiler_params=pltpu.CompilerParams(use_tc_tiling_on_sc=False),
)


def kernel(ids, values, remaps, emb0, emb1, emb2, emb3):
    ids_f = ids.reshape(-1).astype(jnp.int32)
    vals_f = values.reshape(-1).astype(jnp.float32)
    rem = remaps.astype(jnp.int32) + jnp.array(OFFS, jnp.int32)[:, None]
    embcat = jnp.concatenate([emb0, emb1, emb2, emb3], axis=0)
    tab = _normalize_table(embcat)
    out_flat = _sc_call(ids_f, vals_f, rem[0], rem[1], rem[2], rem[3], tab)
    return out_flat.reshape(B, T, D)


# speedup vs baseline: 23.2031x; 1.2740x over previous
import jax
import jax.numpy as jnp
from jax import lax
from jax.experimental import pallas as pl
from jax.experimental.pallas import tpu as pltpu
from jax.experimental.pallas import tpu_sc as plsc

V = 100000
D = 64
NL = 4
B = 4096
T = 50
N = B * T

NC, NS, LANES = 2, 16, 16
NW = NC * NS
PER_W = N // NW
CHUNK = 160
NCHUNK = PER_W // CHUNK

LEVEL_SIZES = (100, 1000, 10000, 100000)
LVL_MULT = (1.0, 0.5, 1.0 / 3.0, 0.25)


def _norm_body(x_ref, o_ref):
    x = x_ref[...]
    s = jnp.sum(x * x, axis=1, keepdims=True)
    inv = lax.rsqrt(jnp.maximum(s, 1e-36))
    o_ref[...] = x * inv


def _normalize_table(emb):
    rows = emb.shape[0]
    r = min(rows, 2000)
    return pl.pallas_call(
        _norm_body,
        grid=(rows // r,),
        in_specs=[pl.BlockSpec((r, D), lambda i: (i, 0))],
        out_specs=pl.BlockSpec((r, D), lambda i: (i, 0)),
        out_shape=jax.ShapeDtypeStruct((rows, D), jnp.float32),
    )(emb)


def _sc_body(ids_h, vals_h, rem0_h, rem1_h, rem2_h, rem3_h,
             tab0_h, tab1_h, tab2_h, tab3_h, out_h,
             idsA, idsB, valsA, valsB, ilA, ilB,
             rA0, rA1, rA2, rA3, rB0, rB1, rB2, rB3,
             outA, outB,
             sem_ids, sem_rem, sem_tabA, sem_tabB, sem_outA, sem_outB):
    cid = lax.axis_index("c")
    sid = lax.axis_index("s")
    wid = sid * NC + cid
    w0 = wid * PER_W
    rems = (rem0_h, rem1_h, rem2_h, rem3_h)
    tabs = (tab0_h, tab1_h, tab2_h, tab3_h)

    bufA = (idsA, valsA, ilA, (rA0, rA1, rA2, rA3), outA, sem_tabA, sem_outA)
    bufB = (idsB, valsB, ilB, (rB0, rB1, rB2, rB3), outB, sem_tabB, sem_outB)

    def issue_ids(ci, buf):
        base = w0 + ci * CHUNK
        pltpu.async_copy(ids_h.at[pl.ds(base, CHUNK)], buf[0], sem_ids)

    def issue_vals(ci, buf):
        base = w0 + ci * CHUNK
        pltpu.async_copy(vals_h.at[pl.ds(base, CHUNK)], buf[1], sem_ids)

    def wait_ids(buf):
        pltpu.make_async_copy(ids_h.at[pl.ds(0, CHUNK)], buf[0], sem_ids).wait()
        pltpu.make_async_copy(vals_h.at[pl.ds(0, CHUNK)], buf[1], sem_ids).wait()

    def issue_rem(buf):
        for l in range(NL):
            pltpu.async_copy(rems[l].at[buf[0]],
                             buf[2].at[pl.ds(l * CHUNK, CHUNK)], sem_rem)

    def wait_rem(buf):
        for l in range(NL):
            pltpu.make_async_copy(rems[l].at[buf[0]],
                                  buf[2].at[pl.ds(l * CHUNK, CHUNK)],
                                  sem_rem).wait()

    def issue_tab(buf):
        for l in range(NL):
            pltpu.async_copy(tabs[l].at[buf[2].at[pl.ds(l * CHUNK, CHUNK)]],
                             buf[3][l], buf[5])

    def wait_tab(buf):
        for l in range(NL):
            pltpu.make_async_copy(
                tabs[l].at[buf[2].at[pl.ds(l * CHUNK, CHUNK)]],
                buf[3][l], buf[5]).wait()

    def issue_out(ci, buf):
        base = w0 + ci * CHUNK
        pltpu.async_copy(buf[4], out_h.at[pl.ds(base, CHUNK)], buf[6])

    def wait_out(buf):
        pltpu.make_async_copy(buf[4], out_h.at[pl.ds(0, CHUNK)], buf[6]).wait()

    def compute(buf):
        _, vals_v, idl, rows, out_v, _, _ = buf
        r0, r1, r2, r3 = rows

        def group(g, c):
            gsl = pl.ds(g * LANES, LANES)
            i0 = idl[pl.ds(0 * CHUNK + g * LANES, LANES)]
            i1 = idl[pl.ds(1 * CHUNK + g * LANES, LANES)]
            i2 = idl[pl.ds(2 * CHUNK + g * LANES, LANES)]
            i3 = idl[pl.ds(3 * CHUNK + g * LANES, LANES)]
            val = vals_v[gsl]
            one = jnp.int32(1)
            z0 = jnp.minimum(jnp.abs(i0), one).astype(jnp.float32)
            z1 = jnp.minimum(jnp.abs(i1), one).astype(jnp.float32)
            z2 = jnp.minimum(jnp.abs(i2), one).astype(jnp.float32)
            z3 = jnp.minimum(jnp.abs(i3), one).astype(jnp.float32)
            m3 = 1.0 - z3
            m2 = 1.0 - z2
            m1 = 1.0 - z1
            sel3 = z3
            sel2 = z2 * m3
            sel1 = z1 * (m2 * m3)
            sel0 = z0 * (m1 * m2 * m3)
            vm1 = val - 1.0
            g0v = (1.0 + sel0 * vm1) * LVL_MULT[0]
            g1v = (1.0 + sel1 * vm1) * LVL_MULT[1]
            g2v = (1.0 + sel2 * vm1) * LVL_MULT[2]
            g3v = (1.0 + sel3 * vm1) * LVL_MULT[3]
            for k in range(LANES):
                t = g * LANES + k
                g0 = g0v[k]
                g1 = g1v[k]
                g2 = g2v[k]
                g3 = g3v[k]
                for j in range(D // LANES):
                    sl = pl.ds(j * LANES, LANES)
                    out_v[t, sl] = (g0 * r0[t, sl] + g1 * r1[t, sl]
                                    + g2 * r2[t, sl] + g3 * r3[t, sl])
            return c

        lax.fori_loop(0, CHUNK // LANES, group, 0)

    def stage(ci, cur, nxt, *, next_chain, next_ids, wait_prev_out):
        if next_chain:
            wait_ids(nxt)
            issue_rem(nxt)
        if next_ids:
            issue_ids(ci + 2, cur)
        if next_chain:
            wait_rem(nxt)
            issue_tab(nxt)
        wait_tab(cur)
        if wait_prev_out:
            wait_out(cur)
        compute(cur)
        issue_out(ci, cur)
        if next_ids:
            issue_vals(ci + 2, cur)

    issue_ids(0, bufA)
    issue_vals(0, bufA)
    wait_ids(bufA)
    issue_rem(bufA)
    issue_ids(1, bufB)
    issue_vals(1, bufB)
    wait_rem(bufA)
    issue_tab(bufA)

    stage(0, bufA, bufB, next_chain=True, next_ids=True, wait_prev_out=False)
    stage(1, bufB, bufA, next_chain=True, next_ids=True, wait_prev_out=False)

    def steady(k, c):
        ci = 2 * k
        stage(ci, bufA, bufB, next_chain=True, next_ids=True,
              wait_prev_out=True)
        stage(ci + 1, bufB, bufA, next_chain=True, next_ids=True,
              wait_prev_out=True)
        return c

    lax.fori_loop(1, NCHUNK // 2 - 1, steady, 0)

    stage(NCHUNK - 2, bufA, bufB, next_chain=True, next_ids=False,
          wait_prev_out=True)
    stage(NCHUNK - 1, bufB, bufA, next_chain=False, next_ids=False,
          wait_prev_out=True)

    wait_out(bufA)
    wait_out(bufB)


_SC_SCRATCH = (
    [pltpu.VMEM((CHUNK,), jnp.int32)] * 2
    + [pltpu.VMEM((CHUNK,), jnp.float32)] * 2
    + [pltpu.VMEM((NL * CHUNK,), jnp.int32)] * 2
    + [pltpu.VMEM((CHUNK, D), jnp.float32)] * 8
    + [pltpu.VMEM((CHUNK, D), jnp.float32)] * 2
    + [pltpu.SemaphoreType.DMA] * 6
)

_sc_call = pl.kernel(
    _sc_body,
    out_type=jax.ShapeDtypeStruct((N, D), jnp.float32),
    mesh=plsc.VectorSubcoreMesh(core_axis_name="c", subcore_axis_name="s"),
    scratch_types=_SC_SCRATCH,
    compiler_params=pltpu.CompilerParams(use_tc_tiling_on_sc=False),
)


def kernel(ids, values, remaps, emb0, emb1, emb2, emb3):
    ids_f = ids.reshape(-1).astype(jnp.int32)
    vals_f = values.reshape(-1).astype(jnp.float32)
    rem = remaps.astype(jnp.int32)
    tabs = [_normalize_table(e) for e in (emb0, emb1, emb2, emb3)]
    out_flat = _sc_call(ids_f, vals_f, rem[0], rem[1], rem[2], rem[3], *tabs)
    return out_flat.reshape(B, T, D)


# speedup vs baseline: 24.3470x; 1.0493x over previous
import jax
import jax.numpy as jnp
from jax import lax
from jax.experimental import pallas as pl
from jax.experimental.pallas import tpu as pltpu
from jax.experimental.pallas import tpu_sc as plsc

V = 100000
D = 64
NL = 4
B = 4096
T = 50
N = B * T

NC, NS, LANES = 2, 16, 16
NW = NC * NS
PER_W = N // NW
CHUNK = 160
NCHUNK = PER_W // CHUNK

LEVEL_SIZES = (100, 1000, 10000, 100000)
LVL_MULT = (1.0, 0.5, 1.0 / 3.0, 0.25)


def _norm_body(x_ref, o_ref):
    x = x_ref[...]
    s = jnp.sum(x * x, axis=1, keepdims=True)
    inv = lax.rsqrt(jnp.maximum(s, 1e-36))
    o_ref[...] = x * inv


def _normalize_table(emb):
    rows = emb.shape[0]
    r = min(rows, 2000)
    return pl.pallas_call(
        _norm_body,
        grid=(rows // r,),
        in_specs=[pl.BlockSpec((r, D), lambda i: (i, 0))],
        out_specs=pl.BlockSpec((r, D), lambda i: (i, 0)),
        out_shape=jax.ShapeDtypeStruct((rows, D), jnp.float32),
    )(emb)


def _sc_body(ids_h, vals_h, rems_h,
             tab0_h, tab1_h, tab2_h, tab3_h, out_h,
             idsA, idsB, valsA, valsB, ilA, ilB,
             rA0, rA1, rA2, rA3, rB0, rB1, rB2, rB3,
             outA, outB,
             sem_ids, sem_rem, sem_tabA, sem_tabB, sem_outA, sem_outB):
    cid = lax.axis_index("c")
    sid = lax.axis_index("s")
    wid = sid * NC + cid
    w0 = wid * PER_W
    tabs = (tab0_h, tab1_h, tab2_h, tab3_h)

    bufA = (idsA, valsA, ilA, (rA0, rA1, rA2, rA3), outA, sem_tabA, sem_outA)
    bufB = (idsB, valsB, ilB, (rB0, rB1, rB2, rB3), outB, sem_tabB, sem_outB)

    def issue_ids(ci, buf):
        base = w0 + ci * CHUNK
        pltpu.async_copy(ids_h.at[pl.ds(base, CHUNK)], buf[0], sem_ids)

    def issue_vals(ci, buf):
        base = w0 + ci * CHUNK
        pltpu.async_copy(vals_h.at[pl.ds(base, CHUNK)], buf[1], sem_ids)

    def wait_ids(buf):
        pltpu.make_async_copy(ids_h.at[pl.ds(0, CHUNK)], buf[0], sem_ids).wait()
        pltpu.make_async_copy(vals_h.at[pl.ds(0, CHUNK)], buf[1], sem_ids).wait()

    def issue_rem(buf):
        for l in range(NL):
            pltpu.async_copy(rems_h.at[l].at[buf[0]],
                             buf[2].at[pl.ds(l * CHUNK, CHUNK)], sem_rem)

    def wait_rem(buf):
        for l in range(NL):
            pltpu.make_async_copy(rems_h.at[l].at[buf[0]],
                                  buf[2].at[pl.ds(l * CHUNK, CHUNK)],
                                  sem_rem).wait()

    def issue_tab(buf):
        for l in range(NL):
            pltpu.async_copy(tabs[l].at[buf[2].at[pl.ds(l * CHUNK, CHUNK)]],
                             buf[3][l], buf[5])

    def wait_tab(buf):
        for l in range(NL):
            pltpu.make_async_copy(
                tabs[l].at[buf[2].at[pl.ds(l * CHUNK, CHUNK)]],
                buf[3][l], buf[5]).wait()

    def issue_out(ci, buf):
        base = (w0 + ci * CHUNK) * D
        pltpu.async_copy(buf[4], out_h.at[pl.ds(base, CHUNK * D)], buf[6])

    def wait_out(buf):
        pltpu.make_async_copy(buf[4], out_h.at[pl.ds(0, CHUNK * D)],
                              buf[6]).wait()

    def compute(buf):
        _, vals_v, idl, rows, out_v, _, _ = buf
        r0, r1, r2, r3 = rows

        def group(g, c):
            gsl = pl.ds(g * LANES, LANES)
            i0 = idl[pl.ds(0 * CHUNK + g * LANES, LANES)]
            i1 = idl[pl.ds(1 * CHUNK + g * LANES, LANES)]
            i2 = idl[pl.ds(2 * CHUNK + g * LANES, LANES)]
            i3 = idl[pl.ds(3 * CHUNK + g * LANES, LANES)]
            val = vals_v[gsl]
            one = jnp.int32(1)
            z0 = jnp.minimum(jnp.abs(i0), one).astype(jnp.float32)
            z1 = jnp.minimum(jnp.abs(i1), one).astype(jnp.float32)
            z2 = jnp.minimum(jnp.abs(i2), one).astype(jnp.float32)
            z3 = jnp.minimum(jnp.abs(i3), one).astype(jnp.float32)
            m3 = 1.0 - z3
            m2 = 1.0 - z2
            m1 = 1.0 - z1
            sel3 = z3
            sel2 = z2 * m3
            sel1 = z1 * (m2 * m3)
            sel0 = z0 * (m1 * m2 * m3)
            vm1 = val - 1.0
            g0v = (1.0 + sel0 * vm1) * LVL_MULT[0]
            g1v = (1.0 + sel1 * vm1) * LVL_MULT[1]
            g2v = (1.0 + sel2 * vm1) * LVL_MULT[2]
            g3v = (1.0 + sel3 * vm1) * LVL_MULT[3]
            for k in range(LANES):
                t = g * LANES + k
                g0 = g0v[k]
                g1 = g1v[k]
                g2 = g2v[k]
                g3 = g3v[k]
                for j in range(D // LANES):
                    sl = pl.ds(j * LANES, LANES)
                    out_v[pl.ds(t * D + j * LANES, LANES)] = (
                        g0 * r0[t, sl] + g1 * r1[t, sl]
                        + g2 * r2[t, sl] + g3 * r3[t, sl])
            return c

        lax.fori_loop(0, CHUNK // LANES, group, 0)

    def stage(ci, cur, nxt, *, next_chain, next_ids, wait_prev_out):
        if next_chain:
            wait_ids(nxt)
            issue_rem(nxt)
        if next_ids:
            issue_ids(ci + 2, cur)
        if next_chain:
            wait_rem(nxt)
            issue_tab(nxt)
        wait_tab(cur)
        if wait_prev_out:
            wait_out(cur)
        compute(cur)
        issue_out(ci, cur)
        if next_ids:
            issue_vals(ci + 2, cur)

    issue_ids(0, bufA)
    issue_vals(0, bufA)
    wait_ids(bufA)
    issue_rem(bufA)
    issue_ids(1, bufB)
    issue_vals(1, bufB)
    wait_rem(bufA)
    issue_tab(bufA)

    stage(0, bufA, bufB, next_chain=True, next_ids=True, wait_prev_out=False)
    stage(1, bufB, bufA, next_chain=True, next_ids=True, wait_prev_out=False)

    def steady(k, c):
        ci = 2 * k
        stage(ci, bufA, bufB, next_chain=True, next_ids=True,
              wait_prev_out=True)
        stage(ci + 1, bufB, bufA, next_chain=True, next_ids=True,
              wait_prev_out=True)
        return c

    lax.fori_loop(1, NCHUNK // 2 - 1, steady, 0)

    stage(NCHUNK - 2, bufA, bufB, next_chain=True, next_ids=False,
          wait_prev_out=True)
    stage(NCHUNK - 1, bufB, bufA, next_chain=False, next_ids=False,
          wait_prev_out=True)

    wait_out(bufA)
    wait_out(bufB)


_SC_SCRATCH = (
    [pltpu.VMEM((CHUNK,), jnp.int32)] * 2
    + [pltpu.VMEM((CHUNK,), jnp.float32)] * 2
    + [pltpu.VMEM((NL * CHUNK,), jnp.int32)] * 2
    + [pltpu.VMEM((CHUNK, D), jnp.float32)] * 8
    + [pltpu.VMEM((CHUNK * D,), jnp.float32)] * 2
    + [pltpu.SemaphoreType.DMA] * 6
)


_sc_call = pl.kernel(
    _sc_body,
    out_type=jax.ShapeDtypeStruct((N * D,), jnp.float32),
    mesh=plsc.VectorSubcoreMesh(core_axis_name="c", subcore_axis_name="s"),
    scratch_types=_SC_SCRATCH,
    compiler_params=pltpu.CompilerParams(use_tc_tiling_on_sc=False),
)


def kernel(ids, values, remaps, emb0, emb1, emb2, emb3):
    ids_f = ids.reshape(-1).astype(jnp.int32)
    vals_f = values.reshape(-1).astype(jnp.float32)
    rem = remaps.astype(jnp.int32)
    tabs = [_normalize_table(e) for e in (emb0, emb1, emb2, emb3)]
    out_flat = _sc_call(ids_f, vals_f, rem, *tabs)
    return out_flat.reshape(B, T, D)


# speedup vs baseline: 25.2531x; 1.0372x over previous
import jax
import jax.numpy as jnp
from jax import lax
from jax.experimental import pallas as pl
from jax.experimental.pallas import tpu as pltpu
from jax.experimental.pallas import tpu_sc as plsc

V = 100000
D = 64
NL = 4
B = 4096
T = 50
N = B * T

NC, NS, LANES = 2, 16, 16
NW = NC * NS
PER_W = N // NW
CHUNK = 160
NCHUNK = PER_W // CHUNK

LEVEL_SIZES = (100, 1000, 10000, 100000)
LVL_MULT = (1.0, 0.5, 1.0 / 3.0, 0.25)


def _norm_body(x_ref, o_ref):
    x = x_ref[...]
    s = jnp.sum(x * x, axis=1, keepdims=True)
    inv = lax.rsqrt(jnp.maximum(s, 1e-36))
    o_ref[...] = x * inv


def _normalize_table(emb):
    rows = emb.shape[0]
    r = min(rows, 5000)
    return pl.pallas_call(
        _norm_body,
        grid=(rows // r,),
        in_specs=[pl.BlockSpec((r, D), lambda i: (i, 0))],
        out_specs=pl.BlockSpec((r, D), lambda i: (i, 0)),
        out_shape=jax.ShapeDtypeStruct((rows, D), jnp.float32),
    )(emb)


def _sc_body(ids_h, vals_h, rems_h,
             tab0_h, tab1_h, tab2_h, tab3_h, out_h,
             idsA, idsB, valsA, valsB, ilA, ilB,
             rA0, rA1, rA2, rA3, rB0, rB1, rB2, rB3,
             outA, outB,
             sem_ids, sem_rem, sem_tabA, sem_tabB, sem_outA, sem_outB):
    cid = lax.axis_index("c")
    sid = lax.axis_index("s")
    wid = sid * NC + cid
    w0 = wid * PER_W
    tabs = (tab0_h, tab1_h, tab2_h, tab3_h)

    bufA = (idsA, valsA, ilA, (rA0, rA1, rA2, rA3), outA, sem_tabA, sem_outA)
    bufB = (idsB, valsB, ilB, (rB0, rB1, rB2, rB3), outB, sem_tabB, sem_outB)

    def issue_ids(ci, buf):
        base = w0 + ci * CHUNK
        pltpu.async_copy(ids_h.at[pl.ds(base, CHUNK)], buf[0], sem_ids)

    def issue_vals(ci, buf):
        base = w0 + ci * CHUNK
        pltpu.async_copy(vals_h.at[pl.ds(base, CHUNK)], buf[1], sem_ids)

    def wait_ids(buf):
        pltpu.make_async_copy(ids_h.at[pl.ds(0, CHUNK)], buf[0], sem_ids).wait()
        pltpu.make_async_copy(vals_h.at[pl.ds(0, CHUNK)], buf[1], sem_ids).wait()

    def issue_rem(buf):
        for l in range(NL):
            pltpu.async_copy(rems_h.at[l].at[buf[0]],
                             buf[2].at[pl.ds(l * CHUNK, CHUNK)], sem_rem)

    def wait_rem(buf):
        for l in range(NL):
            pltpu.make_async_copy(rems_h.at[l].at[buf[0]],
                                  buf[2].at[pl.ds(l * CHUNK, CHUNK)],
                                  sem_rem).wait()

    def issue_tab(buf):
        for l in range(NL):
            pltpu.async_copy(tabs[l].at[buf[2].at[pl.ds(l * CHUNK, CHUNK)]],
                             buf[3][l], buf[5])

    def wait_tab(buf):
        for l in range(NL):
            pltpu.make_async_copy(
                tabs[l].at[buf[2].at[pl.ds(l * CHUNK, CHUNK)]],
                buf[3][l], buf[5]).wait()

    def issue_out(ci, buf):
        base = (w0 + ci * CHUNK) * D
        pltpu.async_copy(buf[4], out_h.at[pl.ds(base, CHUNK * D)], buf[6])

    def wait_out(buf):
        pltpu.make_async_copy(buf[4], out_h.at[pl.ds(0, CHUNK * D)],
                              buf[6]).wait()

    def compute(buf):
        _, vals_v, idl, rows, out_v, _, _ = buf
        r0, r1, r2, r3 = rows

        def group(g, c):
            gsl = pl.ds(g * LANES, LANES)
            i0 = idl[pl.ds(0 * CHUNK + g * LANES, LANES)]
            i1 = idl[pl.ds(1 * CHUNK + g * LANES, LANES)]
            i2 = idl[pl.ds(2 * CHUNK + g * LANES, LANES)]
            i3 = idl[pl.ds(3 * CHUNK + g * LANES, LANES)]
            val = vals_v[gsl]
            one = jnp.int32(1)
            z0 = jnp.minimum(jnp.abs(i0), one).astype(jnp.float32)
            z1 = jnp.minimum(jnp.abs(i1), one).astype(jnp.float32)
            z2 = jnp.minimum(jnp.abs(i2), one).astype(jnp.float32)
            z3 = jnp.minimum(jnp.abs(i3), one).astype(jnp.float32)
            m3 = 1.0 - z3
            m2 = 1.0 - z2
            m1 = 1.0 - z1
            sel3 = z3
            sel2 = z2 * m3
            sel1 = z1 * (m2 * m3)
            sel0 = z0 * (m1 * m2 * m3)
            vm1 = val - 1.0
            g0v = (1.0 + sel0 * vm1) * LVL_MULT[0]
            g1v = (1.0 + sel1 * vm1) * LVL_MULT[1]
            g2v = (1.0 + sel2 * vm1) * LVL_MULT[2]
            g3v = (1.0 + sel3 * vm1) * LVL_MULT[3]
            for k in range(LANES):
                t = g * LANES + k
                g0 = g0v[k]
                g1 = g1v[k]
                g2 = g2v[k]
                g3 = g3v[k]
                for j in range(D // LANES):
                    sl = pl.ds(j * LANES, LANES)
                    out_v[pl.ds(t * D + j * LANES, LANES)] = (
                        g0 * r0[t, sl] + g1 * r1[t, sl]
                        + g2 * r2[t, sl] + g3 * r3[t, sl])
            return c

        lax.fori_loop(0, CHUNK // LANES, group, 0)

    def stage(ci, cur, nxt, *, next_chain, next_ids, wait_prev_out):
        if next_chain:
            wait_ids(nxt)
            issue_rem(nxt)
        if next_ids:
            issue_ids(ci + 2, cur)
        if next_chain:
            wait_rem(nxt)
            issue_tab(nxt)
        wait_tab(cur)
        if wait_prev_out:
            wait_out(cur)
        compute(cur)
        issue_out(ci, cur)
        if next_ids:
            issue_vals(ci + 2, cur)

    issue_ids(0, bufA)
    issue_vals(0, bufA)
    wait_ids(bufA)
    issue_rem(bufA)
    issue_ids(1, bufB)
    issue_vals(1, bufB)
    wait_rem(bufA)
    issue_tab(bufA)

    stage(0, bufA, bufB, next_chain=True, next_ids=True, wait_prev_out=False)
    stage(1, bufB, bufA, next_chain=True, next_ids=True, wait_prev_out=False)

    def steady(k, c):
        ci = 2 * k
        stage(ci, bufA, bufB, next_chain=True, next_ids=True,
              wait_prev_out=True)
        stage(ci + 1, bufB, bufA, next_chain=True, next_ids=True,
              wait_prev_out=True)
        return c

    lax.fori_loop(1, NCHUNK // 2 - 1, steady, 0)

    stage(NCHUNK - 2, bufA, bufB, next_chain=True, next_ids=False,
          wait_prev_out=True)
    stage(NCHUNK - 1, bufB, bufA, next_chain=False, next_ids=False,
          wait_prev_out=True)

    wait_out(bufA)
    wait_out(bufB)


_SC_SCRATCH = (
    [pltpu.VMEM((CHUNK,), jnp.int32)] * 2
    + [pltpu.VMEM((CHUNK,), jnp.float32)] * 2
    + [pltpu.VMEM((NL * CHUNK,), jnp.int32)] * 2
    + [pltpu.VMEM((CHUNK, D), jnp.float32)] * 8
    + [pltpu.VMEM((CHUNK * D,), jnp.float32)] * 2
    + [pltpu.SemaphoreType.DMA] * 6
)


_sc_call = pl.kernel(
    _sc_body,
    out_type=jax.ShapeDtypeStruct((N * D,), jnp.float32),
    mesh=plsc.VectorSubcoreMesh(core_axis_name="c", subcore_axis_name="s"),
    scratch_types=_SC_SCRATCH,
    compiler_params=pltpu.CompilerParams(use_tc_tiling_on_sc=False),
)


def kernel(ids, values, remaps, emb0, emb1, emb2, emb3):
    ids_f = ids.reshape(-1).astype(jnp.int32)
    vals_f = values.reshape(-1).astype(jnp.float32)
    rem = remaps.astype(jnp.int32)
    tabs = [_normalize_table(e) for e in (emb0, emb1, emb2, emb3)]
    out_flat = _sc_call(ids_f, vals_f, rem, *tabs)
    return out_flat.reshape(B, T, D)


# speedup vs baseline: 25.3142x; 1.0024x over previous
import jax
import jax.numpy as jnp
from jax import lax
from jax.experimental import pallas as pl
from jax.experimental.pallas import tpu as pltpu
from jax.experimental.pallas import tpu_sc as plsc

V = 100000
D = 64
NL = 4
B = 4096
T = 50
N = B * T

NC, NS, LANES = 2, 16, 16
NW = NC * NS
PER_W = N // NW
CHUNK = 160
NCHUNK = PER_W // CHUNK

LEVEL_SIZES = (100, 1000, 10000, 100000)
LVL_MULT = (1.0, 0.5, 1.0 / 3.0, 0.25)


def _norm_body(x_ref, o_ref):
    x = x_ref[...]
    s = jnp.sum(x * x, axis=1, keepdims=True)
    inv = lax.rsqrt(jnp.maximum(s, 1e-36))
    o_ref[...] = x * inv


def _normalize_table(emb):
    rows = emb.shape[0]
    r = min(rows, 10000)
    return pl.pallas_call(
        _norm_body,
        grid=(rows // r,),
        in_specs=[pl.BlockSpec((r, D), lambda i: (i, 0))],
        out_specs=pl.BlockSpec((r, D), lambda i: (i, 0)),
        out_shape=jax.ShapeDtypeStruct((rows, D), jnp.float32),
    )(emb)


def _sc_body(ids_h, vals_h, rems_h,
             tab0_h, tab1_h, tab2_h, tab3_h, out_h,
             idsA, idsB, valsA, valsB, ilA, ilB,
             rA0, rA1, rA2, rA3, rB0, rB1, rB2, rB3,
             outA, outB,
             sem_ids, sem_rem, sem_tabA, sem_tabB, sem_outA, sem_outB):
    cid = lax.axis_index("c")
    sid = lax.axis_index("s")
    wid = sid * NC + cid
    w0 = wid * PER_W
    tabs = (tab0_h, tab1_h, tab2_h, tab3_h)

    bufA = (idsA, valsA, ilA, (rA0, rA1, rA2, rA3), outA, sem_tabA, sem_outA)
    bufB = (idsB, valsB, ilB, (rB0, rB1, rB2, rB3), outB, sem_tabB, sem_outB)

    def issue_ids(ci, buf):
        base = w0 + ci * CHUNK
        pltpu.async_copy(ids_h.at[pl.ds(base, CHUNK)], buf[0], sem_ids)

    def issue_vals(ci, buf):
        base = w0 + ci * CHUNK
        pltpu.async_copy(vals_h.at[pl.ds(base, CHUNK)], buf[1], sem_ids)

    def wait_ids(buf):
        pltpu.make_async_copy(ids_h.at[pl.ds(0, CHUNK)], buf[0], sem_ids).wait()
        pltpu.make_async_copy(vals_h.at[pl.ds(0, CHUNK)], buf[1], sem_ids).wait()

    def issue_rem(buf):
        for l in range(NL):
            pltpu.async_copy(rems_h.at[l].at[buf[0]],
                             buf[2].at[pl.ds(l * CHUNK, CHUNK)], sem_rem)

    def wait_rem(buf):
        for l in range(NL):
            pltpu.make_async_copy(rems_h.at[l].at[buf[0]],
                                  buf[2].at[pl.ds(l * CHUNK, CHUNK)],
                                  sem_rem).wait()

    def issue_tab(buf):
        for l in range(NL):
            pltpu.async_copy(tabs[l].at[buf[2].at[pl.ds(l * CHUNK, CHUNK)]],
                             buf[3][l], buf[5])

    def wait_tab(buf):
        for l in range(NL):
            pltpu.make_async_copy(
                tabs[l].at[buf[2].at[pl.ds(l * CHUNK, CHUNK)]],
                buf[3][l], buf[5]).wait()

    def issue_out(ci, buf):
        base = (w0 + ci * CHUNK) * D
        pltpu.async_copy(buf[4], out_h.at[pl.ds(base, CHUNK * D)], buf[6])

    def wait_out(buf):
        pltpu.make_async_copy(buf[4], out_h.at[pl.ds(0, CHUNK * D)],
                              buf[6]).wait()

    def compute(buf):
        _, vals_v, idl, rows, out_v, _, _ = buf
        r0, r1, r2, r3 = rows

        def group(g, c):
            gsl = pl.ds(g * LANES, LANES)
            i0 = idl[pl.ds(0 * CHUNK + g * LANES, LANES)]
            i1 = idl[pl.ds(1 * CHUNK + g * LANES, LANES)]
            i2 = idl[pl.ds(2 * CHUNK + g * LANES, LANES)]
            i3 = idl[pl.ds(3 * CHUNK + g * LANES, LANES)]
            val = vals_v[gsl]
            one = jnp.int32(1)
            z0 = jnp.minimum(jnp.abs(i0), one).astype(jnp.float32)
            z1 = jnp.minimum(jnp.abs(i1), one).astype(jnp.float32)
            z2 = jnp.minimum(jnp.abs(i2), one).astype(jnp.float32)
            z3 = jnp.minimum(jnp.abs(i3), one).astype(jnp.float32)
            m3 = 1.0 - z3
            m2 = 1.0 - z2
            m1 = 1.0 - z1
            sel3 = z3
            sel2 = z2 * m3
            sel1 = z1 * (m2 * m3)
            sel0 = z0 * (m1 * m2 * m3)
            vm1 = val - 1.0
            g0v = (1.0 + sel0 * vm1) * LVL_MULT[0]
            g1v = (1.0 + sel1 * vm1) * LVL_MULT[1]
            g2v = (1.0 + sel2 * vm1) * LVL_MULT[2]
            g3v = (1.0 + sel3 * vm1) * LVL_MULT[3]
            for k in range(LANES):
                t = g * LANES + k
                g0 = g0v[k]
                g1 = g1v[k]
                g2 = g2v[k]
                g3 = g3v[k]
                for j in range(D // LANES):
                    sl = pl.ds(j * LANES, LANES)
                    out_v[pl.ds(t * D + j * LANES, LANES)] = (
                        g0 * r0[t, sl] + g1 * r1[t, sl]
                        + g2 * r2[t, sl] + g3 * r3[t, sl])
            return c

        lax.fori_loop(0, CHUNK // LANES, group, 0)

    def stage(ci, cur, nxt, *, next_chain, next_ids, wait_prev_out):
        if next_chain:
            wait_ids(nxt)
            issue_rem(nxt)
        if next_ids:
            issue_ids(ci + 2, cur)
        if next_chain:
            wait_rem(nxt)
            issue_tab(nxt)
        wait_tab(cur)
        if wait_prev_out:
            wait_out(cur)
        compute(cur)
        issue_out(ci, cur)
        if next_ids:
            issue_vals(ci + 2, cur)

    issue_ids(0, bufA)
    issue_vals(0, bufA)
    wait_ids(bufA)
    issue_rem(bufA)
    issue_ids(1, bufB)
    issue_vals(1, bufB)
    wait_rem(bufA)
    issue_tab(bufA)

    stage(0, bufA, bufB, next_chain=True, next_ids=True, wait_prev_out=False)
    stage(1, bufB, bufA, next_chain=True, next_ids=True, wait_prev_out=False)

    def steady(k, c):
        ci = 2 * k
        stage(ci, bufA, bufB, next_chain=True, next_ids=True,
              wait_prev_out=True)
        stage(ci + 1, bufB, bufA, next_chain=True, next_ids=True,
              wait_prev_out=True)
        return c

    lax.fori_loop(1, NCHUNK // 2 - 1, steady, 0)

    stage(NCHUNK - 2, bufA, bufB, next_chain=True, next_ids=False,
          wait_prev_out=True)
    stage(NCHUNK - 1, bufB, bufA, next_chain=False, next_ids=False,
          wait_prev_out=True)

    wait_out(bufA)
    wait_out(bufB)


_SC_SCRATCH = (
    [pltpu.VMEM((CHUNK,), jnp.int32)] * 2
    + [pltpu.VMEM((CHUNK,), jnp.float32)] * 2
    + [pltpu.VMEM((NL * CHUNK,), jnp.int32)] * 2
    + [pltpu.VMEM((CHUNK, D), jnp.float32)] * 8
    + [pltpu.VMEM((CHUNK * D,), jnp.float32)] * 2
    + [pltpu.SemaphoreType.DMA] * 6
)


_sc_call = pl.kernel(
    _sc_body,
    out_type=jax.ShapeDtypeStruct((N * D,), jnp.float32),
    mesh=plsc.VectorSubcoreMesh(core_axis_name="c", subcore_axis_name="s"),
    scratch_types=_SC_SCRATCH,
    compiler_params=pltpu.CompilerParams(use_tc_tiling_on_sc=False),
)


def kernel(ids, values, remaps, emb0, emb1, emb2, emb3):
    ids_f = ids.reshape(-1).astype(jnp.int32)
    vals_f = values.reshape(-1).astype(jnp.float32)
    rem = remaps.astype(jnp.int32)
    tabs = [_normalize_table(e) for e in (emb0, emb1, emb2, emb3)]
    out_flat = _sc_call(ids_f, vals_f, rem, *tabs)
    return out_flat.reshape(B, T, D)
